# baseline (speedup 1.0000x reference)
import functools

import jax
import jax.numpy as jnp
from jax import lax
from jax.experimental import pallas as pl
from jax.experimental.pallas import tpu as pltpu

NUM_CLUSTERS = 1024
DECAY = 0.999
EPSILON = 1e-06
BN = 512

INTERP = False


def _pq_body(x_ref, means_ref, ids_ref, newm_ref, sumx_ref, cnt_ref):
    nb = pl.program_id(1)
    nnb = pl.num_programs(1)

    @pl.when(nb == 0)
    def _init():
        sumx_ref[...] = jnp.zeros_like(sumx_ref)
        cnt_ref[...] = jnp.zeros_like(cnt_ref)

    xbT = x_ref[0]
    m = means_ref[0]

    mn = jnp.sum(m * m, axis=1, keepdims=True)
    xn = jnp.sum(xbT * xbT, axis=0, keepdims=True)
    prod = lax.dot_general(m, xbT, (((1,), (0,)), ((), ())),
                           preferred_element_type=jnp.float32)
    dists = -2.0 * prod + xn + mn

    dmin = jnp.min(dists, axis=0, keepdims=True)
    kiota = lax.broadcasted_iota(jnp.int32, (NUM_CLUSTERS, BN), 0)
    idmat = jnp.where(dists == dmin, kiota, NUM_CLUSTERS)
    ids = jnp.min(idmat, axis=0, keepdims=True)
    ids_ref[0, 0] = ids

    ohT = (kiota == ids).astype(jnp.float32)
    cnt_ref[...] += jnp.sum(ohT, axis=1, keepdims=True)
    sumx_ref[...] += lax.dot_general(ohT, xbT, (((1,), (1,)), ((), ())),
                                     preferred_element_type=jnp.float32)

    @pl.when(nb == nnb - 1)
    def _fin():
        meansx = sumx_ref[...] / (EPSILON + cnt_ref[...])
        newm_ref[0] = DECAY * m + (1.0 - DECAY) * meansx


def kernel(x, means):
    B, L, H, D = x.shape
    K = means.shape[1]
    N = B * L
    nnb = N // BN

    xT = jnp.transpose(x.reshape(N, H, D), (1, 2, 0))

    ids4, new_means = pl.pallas_call(
        _pq_body,
        grid=(H, nnb),
        in_specs=[
            pl.BlockSpec((1, D, BN), lambda h, nb: (h, 0, nb)),
            pl.BlockSpec((1, K, D), lambda h, nb: (h, 0, 0)),
        ],
        out_specs=[
            pl.BlockSpec((1, 1, 1, BN), lambda h, nb: (h, nb, 0, 0)),
            pl.BlockSpec((1, K, D), lambda h, nb: (h, 0, 0)),
        ],
        out_shape=[
            jax.ShapeDtypeStruct((H, nnb, 1, BN), jnp.int32),
            jax.ShapeDtypeStruct((H, K, D), jnp.float32),
        ],
        scratch_shapes=[
            pltpu.VMEM((K, D), jnp.float32),
            pltpu.VMEM((K, 1), jnp.float32),
        ],
        compiler_params=pltpu.CompilerParams(
            dimension_semantics=("arbitrary", "arbitrary"),
        ),
        interpret=INTERP,
    )(xT, means)

    cluster_ids = jnp.transpose(ids4.reshape(H, N), (1, 0)).reshape(B, L, H)
    return cluster_ids, new_means
